# baseline (speedup 1.0000x reference)
import jax
import jax.numpy as jnp
from jax import lax
from jax.experimental import pallas as pl
from jax.experimental.pallas import tpu as pltpu
from jax.experimental.pallas import tpu_sc as plsc

_MU = 0.02
_BINS = 10
_ALPHA = 0.75
_N = 8388608

_NC, _NS, _L = 2, 16, 16
_NW = _NC * _NS
_CHUNK = 16384
_SC_NCHUNK = 4
_PER_W = _SC_NCHUNK * _CHUNK
_SC_N = _NW * _PER_W
_UB = 8

_TC_N = _N - _SC_N
_BE = 524288
_TC_GRID = _TC_N // _BE
_OFF_BLKS = _SC_N // _BE
_CE = 8192
_NACC = 21


def _sc_body(p_hbm, t_hbm, out_hbm, pb0, pb1, tb0, tb1, vbuf,
             cnt_ref, ls_ref, sp0, sp1, st0, st1):
    wid = lax.axis_index("s") * _NC + lax.axis_index("c")
    base = wid * _PER_W
    for u in range(_UB):
        cnt_ref[pl.ds(u * _L, _L)] = jnp.zeros((_L,), jnp.float32)
        ls_ref[pl.ds(u * _L, _L)] = jnp.zeros((_L,), jnp.float32)

    pbufs = (pb0, pb1)
    tbufs = (tb0, tb1)
    psems = (sp0, sp1)
    tsems = (st0, st1)

    def start(ci, b):
        return (
            pltpu.async_copy(p_hbm.at[pl.ds(base + ci * _CHUNK, _CHUNK)],
                             pbufs[b], psems[b]),
            pltpu.async_copy(t_hbm.at[pl.ds(base + ci * _CHUNK, _CHUNK)],
                             tbufs[b], tsems[b]),
        )

    handles = {0: start(0, 0), 1: start(1, 1)}

    for ci in range(_SC_NCHUNK):
        b = ci & 1
        hp, ht = handles[b]
        hp.wait()
        ht.wait()
        pb, tb = pbufs[b], tbufs[b]

        @plsc.parallel_loop(0, _CHUNK // _L, unroll=_UB)
        def vloop(i):
            p = pb[pl.ds(i * _L, _L)]
            t = tb[pl.ds(i * _L, _L)]
            d = jnp.abs(p - t)
            loss = jnp.where(d < _MU, (0.5 / _MU) * d * d, d - 0.5 * _MU)
            u = 1.0 / (jnp.exp(p + p) + 1.0)
            v = 1.0 / (jnp.exp(t + t) + 1.0)
            m = jnp.abs(v - u) * (2.0 * _BINS)
            idx = m.astype(jnp.int32)
            bank = (i & (_UB - 1)) * _L
            idxf = jnp.minimum(idx, _BINS - 1) + bank
            plsc.addupdate_scatter(ls_ref, [idxf], loss)
            plsc.addupdate_scatter(
                cnt_ref, [idxf],
                jnp.where(idx <= _BINS - 1, 1.0, 0.0))

        nxt = ci + 2
        if nxt < _SC_NCHUNK:
            handles[b] = start(nxt, b)

    cnt = cnt_ref[pl.ds(0, _L)]
    ls = ls_ref[pl.ds(0, _L)]
    for u in range(1, _UB):
        cnt = cnt + cnt_ref[pl.ds(u * _L, _L)]
        ls = ls + ls_ref[pl.ds(u * _L, _L)]
    vbuf[pl.ds(0, _L)] = cnt
    vbuf[pl.ds(_L, _L)] = ls
    pltpu.sync_copy(vbuf, out_hbm.at[wid])


def _red(v):
    acc = v[0:1024]
    for j in range(1, _CE // 1024):
        acc = acc + v[j * 1024:(j + 1) * 1024]
    return acc


def _tc_body(p_ref, t_ref, out_ref, acc_ref):
    step = pl.program_id(0)

    @pl.when(step == 0)
    def _init():
        acc_ref[...] = jnp.zeros_like(acc_ref)

    def chunk(i, accs):
        p = p_ref[pl.ds(i * _CE, _CE)]
        t = t_ref[pl.ds(i * _CE, _CE)]
        d = jnp.abs(p - t)
        loss = jnp.where(d < _MU, (0.5 / _MU) * d * d, d - 0.5 * _MU)
        m = jnp.abs(jnp.tanh(p) - jnp.tanh(t)) * float(_BINS)
        new = list(accs)
        new[0] = accs[0] + _red(loss)
        for k in range(1, _BINS):
            mask = m >= float(k)
            new[k] = accs[k] + _red(jnp.where(mask, loss, 0.0))
            new[_BINS + k] = accs[_BINS + k] + _red(
                jnp.where(mask, 1.0, 0.0))
        new[2 * _BINS] = accs[2 * _BINS] + _red(
            jnp.where(m >= float(_BINS), 1.0, 0.0))
        return tuple(new)

    zero = jnp.zeros((1024,), jnp.float32)
    init = tuple(zero for _ in range(_NACC))
    accs = jax.lax.fori_loop(0, _BE // _CE, chunk, init)
    for k in range(_NACC):
        if k == _BINS:
            continue
        acc_ref[pl.ds(k * 1024, 1024)] += accs[k]

    @pl.when(step == _TC_GRID - 1)
    def _finish():
        for k in range(_NACC):
            if k == _BINS:
                continue
            out_ref[k] = jnp.sum(acc_ref[pl.ds(k * 1024, 1024)])


def _combine_body(tc_ref, sc_ref, out_ref):
    ls_ge = [tc_ref[k] for k in range(_BINS)]
    cnt_ge = [jnp.float32(_TC_N)] + [
        tc_ref[_BINS + k] for k in range(1, _BINS + 1)]
    s = jnp.sum(sc_ref[...], axis=0)
    lanes = jax.lax.iota(jnp.int32, 32)
    cnt_v = jnp.where(lanes < _BINS, s, 0.0)
    ls_v = jnp.where((lanes >= _L) & (lanes < _L + _BINS), s, 0.0)
    for b in range(_BINS):
        cnt_b = cnt_ge[b] - cnt_ge[b + 1]
        ls_b = ls_ge[b] - (ls_ge[b + 1] if b + 1 < _BINS else 0.0)
        cnt_v = jnp.where(lanes == b, cnt_v + cnt_b, cnt_v)
        ls_v = jnp.where(lanes == _L + b, ls_v + ls_b, ls_v)
    tot = jnp.where(lanes < _BINS, jnp.maximum(cnt_v, 1.0), 1.0)
    w = jnp.exp(-_ALPHA * jnp.log(tot))
    wls = jnp.concatenate([w[: _L], w[: _L]]) * ls_v
    out_ref[0, 0] = jnp.sum(wls) * (1.0 / _N)


def kernel(pred, target):
    tc = pl.pallas_call(
        _tc_body,
        grid=(_TC_GRID,),
        in_specs=[
            pl.BlockSpec((_BE,), lambda i: (i + _OFF_BLKS,)),
            pl.BlockSpec((_BE,), lambda i: (i + _OFF_BLKS,)),
        ],
        out_specs=pl.BlockSpec(memory_space=pltpu.SMEM),
        out_shape=jax.ShapeDtypeStruct((_NACC + 1,), jnp.float32),
        scratch_shapes=[pltpu.VMEM(((_NACC + 3) * 1024,), jnp.float32)],
        compiler_params=pltpu.CompilerParams(
            dimension_semantics=("arbitrary",)),
    )(pred, target)

    mesh = plsc.VectorSubcoreMesh(core_axis_name="c", subcore_axis_name="s")
    sc = pl.kernel(
        _sc_body,
        mesh=mesh,
        compiler_params=pltpu.CompilerParams(needs_layout_passes=False),
        out_type=jax.ShapeDtypeStruct((_NW, 2 * _L), jnp.float32),
        scratch_types=[
            pltpu.VMEM((_CHUNK,), jnp.float32),
            pltpu.VMEM((_CHUNK,), jnp.float32),
            pltpu.VMEM((_CHUNK,), jnp.float32),
            pltpu.VMEM((_CHUNK,), jnp.float32),
            pltpu.VMEM((2 * _L,), jnp.float32),
            pltpu.VMEM((_UB * _L,), jnp.float32),
            pltpu.VMEM((_UB * _L,), jnp.float32),
            pltpu.SemaphoreType.DMA,
            pltpu.SemaphoreType.DMA,
            pltpu.SemaphoreType.DMA,
            pltpu.SemaphoreType.DMA,
        ],
    )(pred, target)

    out = pl.pallas_call(
        _combine_body,
        in_specs=[
            pl.BlockSpec(memory_space=pltpu.SMEM),
            pl.BlockSpec(memory_space=pltpu.VMEM),
        ],
        out_specs=pl.BlockSpec(memory_space=pltpu.SMEM),
        out_shape=jax.ShapeDtypeStruct((1, 1), jnp.float32),
    )(tc, sc)
    return out[0, 0]


# speedup vs baseline: 1.1092x; 1.1092x over previous
import jax
import jax.numpy as jnp
from jax import lax
from jax.experimental import pallas as pl
from jax.experimental.pallas import tpu as pltpu
from jax.experimental.pallas import tpu_sc as plsc

_MU = 0.02
_BINS = 10
_ALPHA = 0.75
_N = 8388608

_NC, _NS, _L = 2, 16, 16
_NW = _NC * _NS
_CHUNK = 16384
_SC_NCHUNK = 4
_PER_W = _SC_NCHUNK * _CHUNK
_SC_N = _NW * _PER_W
_UB = 8

_TC_N = _N - _SC_N
_BE = 524288
_TC_GRID = _TC_N // _BE
_OFF_BLKS = _SC_N // _BE
_CE = 8192
_NACC = 21


def _sc_body(p_hbm, t_hbm, out_hbm, pb0, pb1, tb0, tb1, vbuf,
             cnt_ref, ls_ref, sp0, sp1, st0, st1):
    wid = lax.axis_index("s") * _NC + lax.axis_index("c")
    base = wid * _PER_W
    for u in range(_UB):
        cnt_ref[pl.ds(u * _L, _L)] = jnp.zeros((_L,), jnp.float32)
        ls_ref[pl.ds(u * _L, _L)] = jnp.zeros((_L,), jnp.float32)

    pbufs = (pb0, pb1)
    tbufs = (tb0, tb1)
    psems = (sp0, sp1)
    tsems = (st0, st1)

    def start(ci, b):
        return (
            pltpu.async_copy(p_hbm.at[pl.ds(base + ci * _CHUNK, _CHUNK)],
                             pbufs[b], psems[b]),
            pltpu.async_copy(t_hbm.at[pl.ds(base + ci * _CHUNK, _CHUNK)],
                             tbufs[b], tsems[b]),
        )

    handles = {0: start(0, 0), 1: start(1, 1)}

    for ci in range(_SC_NCHUNK):
        b = ci & 1
        hp, ht = handles[b]
        hp.wait()
        ht.wait()
        pb, tb = pbufs[b], tbufs[b]

        @plsc.parallel_loop(0, _CHUNK // _L, unroll=_UB)
        def vloop(i):
            p = pb[pl.ds(i * _L, _L)]
            t = tb[pl.ds(i * _L, _L)]
            d = jnp.abs(p - t)
            loss = jnp.where(d < _MU, (0.5 / _MU) * d * d, d - 0.5 * _MU)
            u = 1.0 / (jnp.exp(p + p) + 1.0)
            v = 1.0 / (jnp.exp(t + t) + 1.0)
            m = jnp.abs(v - u) * (2.0 * _BINS)
            idx = m.astype(jnp.int32)
            bank = (i & (_UB - 1)) * _L
            idxf = jnp.minimum(idx, _BINS - 1) + bank
            plsc.addupdate_scatter(ls_ref, [idxf], loss)
            plsc.addupdate_scatter(
                cnt_ref, [idxf],
                jnp.where(idx <= _BINS - 1, 1.0, 0.0))

        nxt = ci + 2
        if nxt < _SC_NCHUNK:
            handles[b] = start(nxt, b)

    cnt = cnt_ref[pl.ds(0, _L)]
    ls = ls_ref[pl.ds(0, _L)]
    for u in range(1, _UB):
        cnt = cnt + cnt_ref[pl.ds(u * _L, _L)]
        ls = ls + ls_ref[pl.ds(u * _L, _L)]
    vbuf[pl.ds(0, _L)] = cnt
    vbuf[pl.ds(_L, _L)] = ls
    pltpu.sync_copy(vbuf, out_hbm.at[wid])


def _red(v):
    acc = v[0:1024]
    for j in range(1, _CE // 1024):
        acc = acc + v[j * 1024:(j + 1) * 1024]
    return acc


def _tc_body(p_ref, t_ref, out_ref, acc_ref):
    step = pl.program_id(0)

    @pl.when(step == 0)
    def _init():
        acc_ref[...] = jnp.zeros_like(acc_ref)

    def chunk(i, accs):
        p = p_ref[pl.ds(i * _CE, _CE)]
        t = t_ref[pl.ds(i * _CE, _CE)]
        d = jnp.abs(p - t)
        loss = jnp.where(d < _MU, (0.5 / _MU) * d * d, d - 0.5 * _MU)
        m = jnp.abs(jnp.tanh(p) - jnp.tanh(t)) * float(_BINS)
        new = list(accs)
        new[0] = accs[0] + _red(loss)
        for k in range(1, _BINS):
            mask = m >= float(k)
            new[k] = accs[k] + _red(jnp.where(mask, loss, 0.0))
            new[_BINS + k] = accs[_BINS + k] + _red(
                jnp.where(mask, 1.0, 0.0))
        new[2 * _BINS] = accs[2 * _BINS] + _red(
            jnp.where(m >= float(_BINS), 1.0, 0.0))
        return tuple(new)

    zero = jnp.zeros((1024,), jnp.float32)
    init = tuple(zero for _ in range(_NACC))
    accs = jax.lax.fori_loop(0, _BE // _CE, chunk, init, unroll=2)
    for k in range(_NACC):
        if k == _BINS:
            continue
        acc_ref[pl.ds(k * 1024, 1024)] += accs[k]

    @pl.when(step == _TC_GRID - 1)
    def _finish():
        for k in range(_NACC):
            if k == _BINS:
                continue
            out_ref[k] = jnp.sum(acc_ref[pl.ds(k * 1024, 1024)])


def _combine_body(tc_ref, sc_ref, out_ref):
    ls_ge = [tc_ref[k] for k in range(_BINS)]
    cnt_ge = [jnp.float32(_TC_N)] + [
        tc_ref[_BINS + k] for k in range(1, _BINS + 1)]
    s = jnp.sum(sc_ref[...], axis=0)
    lanes = jax.lax.iota(jnp.int32, 32)
    cnt_v = jnp.where(lanes < _BINS, s, 0.0)
    ls_v = jnp.where((lanes >= _L) & (lanes < _L + _BINS), s, 0.0)
    for b in range(_BINS):
        cnt_b = cnt_ge[b] - cnt_ge[b + 1]
        ls_b = ls_ge[b] - (ls_ge[b + 1] if b + 1 < _BINS else 0.0)
        cnt_v = jnp.where(lanes == b, cnt_v + cnt_b, cnt_v)
        ls_v = jnp.where(lanes == _L + b, ls_v + ls_b, ls_v)
    tot = jnp.where(lanes < _BINS, jnp.maximum(cnt_v, 1.0), 1.0)
    w = jnp.exp(-_ALPHA * jnp.log(tot))
    wls = jnp.concatenate([w[: _L], w[: _L]]) * ls_v
    out_ref[0, 0] = jnp.sum(wls) * (1.0 / _N)


def kernel(pred, target):
    tc = pl.pallas_call(
        _tc_body,
        grid=(_TC_GRID,),
        in_specs=[
            pl.BlockSpec((_BE,), lambda i: (i + _OFF_BLKS,)),
            pl.BlockSpec((_BE,), lambda i: (i + _OFF_BLKS,)),
        ],
        out_specs=pl.BlockSpec(memory_space=pltpu.SMEM),
        out_shape=jax.ShapeDtypeStruct((_NACC + 1,), jnp.float32),
        scratch_shapes=[pltpu.VMEM(((_NACC + 3) * 1024,), jnp.float32)],
        compiler_params=pltpu.CompilerParams(
            dimension_semantics=("arbitrary",)),
    )(pred, target)

    mesh = plsc.VectorSubcoreMesh(core_axis_name="c", subcore_axis_name="s")
    sc = pl.kernel(
        _sc_body,
        mesh=mesh,
        compiler_params=pltpu.CompilerParams(needs_layout_passes=False),
        out_type=jax.ShapeDtypeStruct((_NW, 2 * _L), jnp.float32),
        scratch_types=[
            pltpu.VMEM((_CHUNK,), jnp.float32),
            pltpu.VMEM((_CHUNK,), jnp.float32),
            pltpu.VMEM((_CHUNK,), jnp.float32),
            pltpu.VMEM((_CHUNK,), jnp.float32),
            pltpu.VMEM((2 * _L,), jnp.float32),
            pltpu.VMEM((_UB * _L,), jnp.float32),
            pltpu.VMEM((_UB * _L,), jnp.float32),
            pltpu.SemaphoreType.DMA,
            pltpu.SemaphoreType.DMA,
            pltpu.SemaphoreType.DMA,
            pltpu.SemaphoreType.DMA,
        ],
    )(pred, target)

    out = pl.pallas_call(
        _combine_body,
        in_specs=[
            pl.BlockSpec(memory_space=pltpu.SMEM),
            pl.BlockSpec(memory_space=pltpu.VMEM),
        ],
        out_specs=pl.BlockSpec(memory_space=pltpu.SMEM),
        out_shape=jax.ShapeDtypeStruct((1, 1), jnp.float32),
    )(tc, sc)
    return out[0, 0]


# speedup vs baseline: 1.1146x; 1.0049x over previous
import jax
import jax.numpy as jnp
from jax import lax
from jax.experimental import pallas as pl
from jax.experimental.pallas import tpu as pltpu
from jax.experimental.pallas import tpu_sc as plsc

_MU = 0.02
_BINS = 10
_ALPHA = 0.75
_N = 8388608

_NC, _NS, _L = 2, 16, 16
_NW = _NC * _NS
_CHUNK = 16384
_SC_NCHUNK = 4
_PER_W = _SC_NCHUNK * _CHUNK
_SC_N = _NW * _PER_W
_UB = 8

_TC_N = _N - _SC_N
_BE = 524288
_TC_GRID = _TC_N // _BE
_OFF_BLKS = _SC_N // _BE
_CE = 8192
_NACC = 21


def _sc_body(p_hbm, t_hbm, out_hbm, pb0, pb1, tb0, tb1, vbuf,
             cnt_ref, ls_ref, sp0, sp1, st0, st1):
    wid = lax.axis_index("s") * _NC + lax.axis_index("c")
    base = wid * _PER_W
    for u in range(_UB):
        cnt_ref[pl.ds(u * _L, _L)] = jnp.zeros((_L,), jnp.float32)
        ls_ref[pl.ds(u * _L, _L)] = jnp.zeros((_L,), jnp.float32)

    pbufs = (pb0, pb1)
    tbufs = (tb0, tb1)
    psems = (sp0, sp1)
    tsems = (st0, st1)

    def start(ci, b):
        return (
            pltpu.async_copy(p_hbm.at[pl.ds(base + ci * _CHUNK, _CHUNK)],
                             pbufs[b], psems[b]),
            pltpu.async_copy(t_hbm.at[pl.ds(base + ci * _CHUNK, _CHUNK)],
                             tbufs[b], tsems[b]),
        )

    handles = {0: start(0, 0), 1: start(1, 1)}

    for ci in range(_SC_NCHUNK):
        b = ci & 1
        hp, ht = handles[b]
        hp.wait()
        ht.wait()
        pb, tb = pbufs[b], tbufs[b]

        @plsc.parallel_loop(0, _CHUNK // _L, unroll=_UB)
        def vloop(i):
            p = pb[pl.ds(i * _L, _L)]
            t = tb[pl.ds(i * _L, _L)]
            d = jnp.abs(p - t)
            loss = jnp.where(d < _MU, (0.5 / _MU) * d * d, d - 0.5 * _MU)
            u = 1.0 / (jnp.exp(p + p) + 1.0)
            v = 1.0 / (jnp.exp(t + t) + 1.0)
            m = jnp.abs(v - u) * (2.0 * _BINS)
            idx = m.astype(jnp.int32)
            bank = (i & (_UB - 1)) * _L
            idxf = jnp.minimum(idx, _BINS - 1) + bank
            plsc.addupdate_scatter(ls_ref, [idxf], loss)
            plsc.addupdate_scatter(
                cnt_ref, [idxf],
                jnp.where(idx <= _BINS - 1, 1.0, 0.0))

        nxt = ci + 2
        if nxt < _SC_NCHUNK:
            handles[b] = start(nxt, b)

    cnt = cnt_ref[pl.ds(0, _L)]
    ls = ls_ref[pl.ds(0, _L)]
    for u in range(1, _UB):
        cnt = cnt + cnt_ref[pl.ds(u * _L, _L)]
        ls = ls + ls_ref[pl.ds(u * _L, _L)]
    vbuf[pl.ds(0, _L)] = cnt
    vbuf[pl.ds(_L, _L)] = ls
    pltpu.sync_copy(vbuf, out_hbm.at[wid])


def _red(v):
    acc = v[0:1024]
    for j in range(1, _CE // 1024):
        acc = acc + v[j * 1024:(j + 1) * 1024]
    return acc


def _tc_body(p_ref, t_ref, out_ref, acc_ref):
    step = pl.program_id(0)

    @pl.when(step == 0)
    def _init():
        acc_ref[...] = jnp.zeros_like(acc_ref)

    def chunk(i, accs):
        p = p_ref[pl.ds(i * _CE, _CE)]
        t = t_ref[pl.ds(i * _CE, _CE)]
        d = jnp.abs(p - t)
        loss = jnp.where(d < _MU, (0.5 / _MU) * d * d, d - 0.5 * _MU)
        m = jnp.abs(jnp.tanh(p) - jnp.tanh(t)) * float(_BINS)
        new = list(accs)
        new[0] = accs[0] + _red(loss)
        for k in range(1, _BINS):
            mask = m >= float(k)
            new[k] = accs[k] + _red(jnp.where(mask, loss, 0.0))
            new[_BINS + k] = accs[_BINS + k] + _red(
                jnp.where(mask, 1.0, 0.0))
        new[2 * _BINS] = accs[2 * _BINS] + _red(
            jnp.where(m >= float(_BINS), 1.0, 0.0))
        return tuple(new)

    zero = jnp.zeros((1024,), jnp.float32)
    init = tuple(zero for _ in range(_NACC))
    accs = jax.lax.fori_loop(0, _BE // _CE, chunk, init, unroll=4)
    for k in range(_NACC):
        if k == _BINS:
            continue
        acc_ref[pl.ds(k * 1024, 1024)] += accs[k]

    @pl.when(step == _TC_GRID - 1)
    def _finish():
        for k in range(_NACC):
            if k == _BINS:
                continue
            out_ref[k] = jnp.sum(acc_ref[pl.ds(k * 1024, 1024)])


def _combine_body(tc_ref, sc_ref, out_ref):
    ls_ge = [tc_ref[k] for k in range(_BINS)]
    cnt_ge = [jnp.float32(_TC_N)] + [
        tc_ref[_BINS + k] for k in range(1, _BINS + 1)]
    s = jnp.sum(sc_ref[...], axis=0)
    lanes = jax.lax.iota(jnp.int32, 32)
    cnt_v = jnp.where(lanes < _BINS, s, 0.0)
    ls_v = jnp.where((lanes >= _L) & (lanes < _L + _BINS), s, 0.0)
    for b in range(_BINS):
        cnt_b = cnt_ge[b] - cnt_ge[b + 1]
        ls_b = ls_ge[b] - (ls_ge[b + 1] if b + 1 < _BINS else 0.0)
        cnt_v = jnp.where(lanes == b, cnt_v + cnt_b, cnt_v)
        ls_v = jnp.where(lanes == _L + b, ls_v + ls_b, ls_v)
    tot = jnp.where(lanes < _BINS, jnp.maximum(cnt_v, 1.0), 1.0)
    w = jnp.exp(-_ALPHA * jnp.log(tot))
    wls = jnp.concatenate([w[: _L], w[: _L]]) * ls_v
    out_ref[0, 0] = jnp.sum(wls) * (1.0 / _N)


def kernel(pred, target):
    tc = pl.pallas_call(
        _tc_body,
        grid=(_TC_GRID,),
        in_specs=[
            pl.BlockSpec((_BE,), lambda i: (i + _OFF_BLKS,)),
            pl.BlockSpec((_BE,), lambda i: (i + _OFF_BLKS,)),
        ],
        out_specs=pl.BlockSpec(memory_space=pltpu.SMEM),
        out_shape=jax.ShapeDtypeStruct((_NACC + 1,), jnp.float32),
        scratch_shapes=[pltpu.VMEM(((_NACC + 3) * 1024,), jnp.float32)],
        compiler_params=pltpu.CompilerParams(
            dimension_semantics=("arbitrary",)),
    )(pred, target)

    mesh = plsc.VectorSubcoreMesh(core_axis_name="c", subcore_axis_name="s")
    sc = pl.kernel(
        _sc_body,
        mesh=mesh,
        compiler_params=pltpu.CompilerParams(needs_layout_passes=False),
        out_type=jax.ShapeDtypeStruct((_NW, 2 * _L), jnp.float32),
        scratch_types=[
            pltpu.VMEM((_CHUNK,), jnp.float32),
            pltpu.VMEM((_CHUNK,), jnp.float32),
            pltpu.VMEM((_CHUNK,), jnp.float32),
            pltpu.VMEM((_CHUNK,), jnp.float32),
            pltpu.VMEM((2 * _L,), jnp.float32),
            pltpu.VMEM((_UB * _L,), jnp.float32),
            pltpu.VMEM((_UB * _L,), jnp.float32),
            pltpu.SemaphoreType.DMA,
            pltpu.SemaphoreType.DMA,
            pltpu.SemaphoreType.DMA,
            pltpu.SemaphoreType.DMA,
        ],
    )(pred, target)

    out = pl.pallas_call(
        _combine_body,
        in_specs=[
            pl.BlockSpec(memory_space=pltpu.SMEM),
            pl.BlockSpec(memory_space=pltpu.VMEM),
        ],
        out_specs=pl.BlockSpec(memory_space=pltpu.SMEM),
        out_shape=jax.ShapeDtypeStruct((1, 1), jnp.float32),
    )(tc, sc)
    return out[0, 0]


# speedup vs baseline: 1.1163x; 1.0015x over previous
import jax
import jax.numpy as jnp
from jax import lax
from jax.experimental import pallas as pl
from jax.experimental.pallas import tpu as pltpu
from jax.experimental.pallas import tpu_sc as plsc

_MU = 0.02
_BINS = 10
_ALPHA = 0.75
_N = 8388608

_NC, _NS, _L = 2, 16, 16
_NW = _NC * _NS
_CHUNK = 16384
_SC_NCHUNK = 4
_PER_W = _SC_NCHUNK * _CHUNK
_SC_N = _NW * _PER_W
_UB = 8

_TC_N = _N - _SC_N
_BE = 524288
_TC_GRID = _TC_N // _BE
_OFF_BLKS = _SC_N // _BE
_CE = 8192
_NACC = 21


def _sc_body(p_hbm, t_hbm, out_hbm, pb0, pb1, tb0, tb1, vbuf,
             cnt_ref, ls_ref, sp0, sp1, st0, st1):
    wid = lax.axis_index("s") * _NC + lax.axis_index("c")
    base = wid * _PER_W
    for u in range(_UB):
        cnt_ref[pl.ds(u * _L, _L)] = jnp.zeros((_L,), jnp.float32)
        ls_ref[pl.ds(u * _L, _L)] = jnp.zeros((_L,), jnp.float32)

    pbufs = (pb0, pb1)
    tbufs = (tb0, tb1)
    psems = (sp0, sp1)
    tsems = (st0, st1)

    def start(ci, b):
        return (
            pltpu.async_copy(p_hbm.at[pl.ds(base + ci * _CHUNK, _CHUNK)],
                             pbufs[b], psems[b]),
            pltpu.async_copy(t_hbm.at[pl.ds(base + ci * _CHUNK, _CHUNK)],
                             tbufs[b], tsems[b]),
        )

    handles = {0: start(0, 0), 1: start(1, 1)}

    for ci in range(_SC_NCHUNK):
        b = ci & 1
        hp, ht = handles[b]
        hp.wait()
        ht.wait()
        pb, tb = pbufs[b], tbufs[b]

        @plsc.parallel_loop(0, _CHUNK // _L, unroll=_UB)
        def vloop(i):
            p = pb[pl.ds(i * _L, _L)]
            t = tb[pl.ds(i * _L, _L)]
            d = jnp.abs(p - t)
            loss = jnp.where(d < _MU, (0.5 / _MU) * d * d, d - 0.5 * _MU)
            u = 1.0 / (jnp.exp(p + p) + 1.0)
            v = 1.0 / (jnp.exp(t + t) + 1.0)
            m = jnp.abs(v - u) * (2.0 * _BINS)
            idx = m.astype(jnp.int32)
            bank = (i & (_UB - 1)) * _L
            idxf = jnp.minimum(idx, _BINS - 1) + bank
            plsc.addupdate_scatter(ls_ref, [idxf], loss)
            plsc.addupdate_scatter(
                cnt_ref, [idxf],
                jnp.where(idx <= _BINS - 1, 1.0, 0.0))

        nxt = ci + 2
        if nxt < _SC_NCHUNK:
            handles[b] = start(nxt, b)

    cnt = cnt_ref[pl.ds(0, _L)]
    ls = ls_ref[pl.ds(0, _L)]
    for u in range(1, _UB):
        cnt = cnt + cnt_ref[pl.ds(u * _L, _L)]
        ls = ls + ls_ref[pl.ds(u * _L, _L)]
    vbuf[pl.ds(0, _L)] = cnt
    vbuf[pl.ds(_L, _L)] = ls
    pltpu.sync_copy(vbuf, out_hbm.at[wid])


def _red(v):
    acc = v[0:1024]
    for j in range(1, _CE // 1024):
        acc = acc + v[j * 1024:(j + 1) * 1024]
    return acc


def _tc_body(p_ref, t_ref, out_ref, acc_ref):
    step = pl.program_id(0)

    @pl.when(step == 0)
    def _init():
        acc_ref[...] = jnp.zeros_like(acc_ref)

    def chunk(i, accs):
        p = p_ref[pl.ds(i * _CE, _CE)]
        t = t_ref[pl.ds(i * _CE, _CE)]
        d = jnp.abs(p - t)
        loss = jnp.where(d < _MU, (0.5 / _MU) * d * d, d - 0.5 * _MU)
        m = jnp.abs(jnp.tanh(p) - jnp.tanh(t)) * float(_BINS)
        new = list(accs)
        new[0] = accs[0] + _red(loss)
        for k in range(1, _BINS):
            mask = m >= float(k)
            new[k] = accs[k] + _red(jnp.where(mask, loss, 0.0))
            new[_BINS + k] = accs[_BINS + k] + _red(
                jnp.where(mask, 1.0, 0.0))
        new[2 * _BINS] = accs[2 * _BINS] + _red(
            jnp.where(m >= float(_BINS), 1.0, 0.0))
        return tuple(new)

    zero = jnp.zeros((1024,), jnp.float32)
    init = tuple(zero for _ in range(_NACC))
    accs = jax.lax.fori_loop(0, _BE // _CE, chunk, init, unroll=8)
    for k in range(_NACC):
        if k == _BINS:
            continue
        acc_ref[pl.ds(k * 1024, 1024)] += accs[k]

    @pl.when(step == _TC_GRID - 1)
    def _finish():
        for k in range(_NACC):
            if k == _BINS:
                continue
            out_ref[k] = jnp.sum(acc_ref[pl.ds(k * 1024, 1024)])


def _combine_body(tc_ref, sc_ref, out_ref):
    ls_ge = [tc_ref[k] for k in range(_BINS)]
    cnt_ge = [jnp.float32(_TC_N)] + [
        tc_ref[_BINS + k] for k in range(1, _BINS + 1)]
    s = jnp.sum(sc_ref[...], axis=0)
    lanes = jax.lax.iota(jnp.int32, 32)
    cnt_v = jnp.where(lanes < _BINS, s, 0.0)
    ls_v = jnp.where((lanes >= _L) & (lanes < _L + _BINS), s, 0.0)
    for b in range(_BINS):
        cnt_b = cnt_ge[b] - cnt_ge[b + 1]
        ls_b = ls_ge[b] - (ls_ge[b + 1] if b + 1 < _BINS else 0.0)
        cnt_v = jnp.where(lanes == b, cnt_v + cnt_b, cnt_v)
        ls_v = jnp.where(lanes == _L + b, ls_v + ls_b, ls_v)
    tot = jnp.where(lanes < _BINS, jnp.maximum(cnt_v, 1.0), 1.0)
    w = jnp.exp(-_ALPHA * jnp.log(tot))
    wls = jnp.concatenate([w[: _L], w[: _L]]) * ls_v
    out_ref[0, 0] = jnp.sum(wls) * (1.0 / _N)


def kernel(pred, target):
    tc = pl.pallas_call(
        _tc_body,
        grid=(_TC_GRID,),
        in_specs=[
            pl.BlockSpec((_BE,), lambda i: (i + _OFF_BLKS,)),
            pl.BlockSpec((_BE,), lambda i: (i + _OFF_BLKS,)),
        ],
        out_specs=pl.BlockSpec(memory_space=pltpu.SMEM),
        out_shape=jax.ShapeDtypeStruct((_NACC + 1,), jnp.float32),
        scratch_shapes=[pltpu.VMEM(((_NACC + 3) * 1024,), jnp.float32)],
        compiler_params=pltpu.CompilerParams(
            dimension_semantics=("arbitrary",)),
    )(pred, target)

    mesh = plsc.VectorSubcoreMesh(core_axis_name="c", subcore_axis_name="s")
    sc = pl.kernel(
        _sc_body,
        mesh=mesh,
        compiler_params=pltpu.CompilerParams(needs_layout_passes=False),
        out_type=jax.ShapeDtypeStruct((_NW, 2 * _L), jnp.float32),
        scratch_types=[
            pltpu.VMEM((_CHUNK,), jnp.float32),
            pltpu.VMEM((_CHUNK,), jnp.float32),
            pltpu.VMEM((_CHUNK,), jnp.float32),
            pltpu.VMEM((_CHUNK,), jnp.float32),
            pltpu.VMEM((2 * _L,), jnp.float32),
            pltpu.VMEM((_UB * _L,), jnp.float32),
            pltpu.VMEM((_UB * _L,), jnp.float32),
            pltpu.SemaphoreType.DMA,
            pltpu.SemaphoreType.DMA,
            pltpu.SemaphoreType.DMA,
            pltpu.SemaphoreType.DMA,
        ],
    )(pred, target)

    out = pl.pallas_call(
        _combine_body,
        in_specs=[
            pl.BlockSpec(memory_space=pltpu.SMEM),
            pl.BlockSpec(memory_space=pltpu.VMEM),
        ],
        out_specs=pl.BlockSpec(memory_space=pltpu.SMEM),
        out_shape=jax.ShapeDtypeStruct((1, 1), jnp.float32),
    )(tc, sc)
    return out[0, 0]
